# Initial kernel scaffold; baseline (speedup 1.0000x reference)
import jax
import jax.numpy as jnp
from jax.experimental import pallas as pl


def kernel(X, edge_index, W, b):
    raise NotImplementedError("write your pallas kernel here")


# speedup vs baseline: 24.5924x; 24.5924x over previous
import functools

import jax
import jax.numpy as jnp
from jax import lax
from jax.experimental import pallas as pl
from jax.experimental.pallas import tpu as pltpu
from jax.experimental.pallas import tpu_sc as plsc

NC = 2
NS = 16
CHUNK = 128


def _mlp_body(x_ref, w_ref, b_ref, o_ref):
    o_ref[...] = jnp.maximum(
        jnp.dot(x_ref[...], w_ref[...], preferred_element_type=jnp.float32)
        + b_ref[...],
        0.0,
    )


def _mlp(x2d, W, b):
    M, D = x2d.shape
    BM = 2000
    assert M % BM == 0
    return pl.pallas_call(
        _mlp_body,
        grid=(M // BM,),
        in_specs=[
            pl.BlockSpec((BM, D), lambda i: (i, 0)),
            pl.BlockSpec((D, D), lambda i: (0, 0)),
            pl.BlockSpec((1, D), lambda i: (0, 0)),
        ],
        out_specs=pl.BlockSpec((BM, D), lambda i: (i, 0)),
        out_shape=jax.ShapeDtypeStruct((M, D), jnp.float32),
    )(x2d, W, b.reshape(1, D))


@functools.partial(jax.jit, static_argnames=("B", "N", "D", "E_pad", "N_pad"))
def _sc_scatter(tx2d, src_g, dst_p, *, B, N, D, E_pad, N_pad):
    per_tile = E_pad // NS
    n_chunks = per_tile // CHUNK
    rows_per_tile = N_pad // NS
    out_rows = (N // NS) // 8 * 8
    out_rem = N - out_rows * NS
    b_per_core = B // NC

    mesh = plsc.VectorSubcoreMesh(
        core_axis_name="c", subcore_axis_name="s",
        num_cores=NC, num_subcores=NS,
    )

    @functools.partial(
        pl.kernel,
        out_type=jax.ShapeDtypeStruct((B * N, D), jnp.float32),
        mesh=mesh,
        scratch_types=[
            pltpu.VMEM((CHUNK,), jnp.int32),
            pltpu.VMEM((CHUNK,), jnp.int32),
            pltpu.VMEM((CHUNK, D), jnp.float32),
            pltpu.VMEM((64, D), jnp.float32),
            pltpu.VMEM_SHARED((N_pad, D), jnp.float32),
            pltpu.SemaphoreType.DMA,
        ],
    )
    def sc_kernel(tx_hbm, srcg_hbm, dst_hbm, out_hbm,
                  src_v, dst_v, rows_v, zeros_v, acc_sh, sem):
        c = lax.axis_index("c")
        s = lax.axis_index("s")

        zero16 = jnp.zeros((16,), jnp.float32)

        def zrow(i, carry):
            for j in range(D // 16):
                zeros_v[i, pl.ds(j * 16, 16)] = zero16
            return carry

        lax.fori_loop(0, 64, zrow, 0)

        for bi in range(b_per_core):
            b = c * b_per_core + bi

            def clr(j, carry):
                pltpu.sync_copy(
                    zeros_v,
                    acc_sh.at[pl.ds(s * rows_per_tile + j * 64, 64)])
                return carry

            lax.fori_loop(0, rows_per_tile // 64, clr, 0)
            plsc.subcore_barrier()

            e_base = b * E_pad + s * per_tile
            d_base = s * per_tile

            def chunk_body(g, carry):
                pltpu.sync_copy(srcg_hbm.at[pl.ds(e_base + g * CHUNK, CHUNK)],
                                src_v)
                pltpu.sync_copy(dst_hbm.at[pl.ds(d_base + g * CHUNK, CHUNK)],
                                dst_v)
                pltpu.async_copy(tx_hbm.at[src_v], rows_v, sem).wait()
                pltpu.sync_copy(rows_v, acc_sh.at[dst_v], add=True)
                return carry

            lax.fori_loop(0, n_chunks, chunk_body, 0)
            plsc.subcore_barrier()

            row_lo = s * out_rows
            pltpu.sync_copy(acc_sh.at[pl.ds(row_lo, out_rows)],
                            out_hbm.at[pl.ds(b * N + row_lo, out_rows)])
            if out_rem:
                @pl.when(s == NS - 1)
                def _():
                    lo = out_rows * NS
                    pltpu.sync_copy(
                        acc_sh.at[pl.ds(lo, out_rem)],
                        out_hbm.at[pl.ds(b * N + lo, out_rem)])
            plsc.subcore_barrier()

    return sc_kernel(tx2d, src_g, dst_p)


def kernel(X, edge_index, W, b):
    B, N, D = X.shape
    E = edge_index.shape[1]
    grain = NS * CHUNK
    E_pad = ((E + grain - 1) // grain) * grain
    N_pad = ((N + 1 + NS * 64 - 1) // (NS * 64)) * (NS * 64)

    src = edge_index[0]
    dst = edge_index[1]
    pad = E_pad - E
    if pad:
        src = jnp.concatenate([src, jnp.zeros((pad,), jnp.int32)])
        dst = jnp.concatenate([dst, jnp.full((pad,), N, jnp.int32)])
    src_g = (src[None, :]
             + (jnp.arange(B, dtype=jnp.int32) * N)[:, None]).reshape(-1)

    tx = _mlp(X.reshape(B * N, D), W, b)
    out2d = _sc_scatter(tx, src_g, dst, B=B, N=N, D=D,
                        E_pad=E_pad, N_pad=N_pad)
    return out2d.reshape(B, N, D)


# speedup vs baseline: 26.4593x; 1.0759x over previous
import functools

import jax
import jax.numpy as jnp
from jax import lax
from jax.experimental import pallas as pl
from jax.experimental.pallas import tpu as pltpu
from jax.experimental.pallas import tpu_sc as plsc

NC = 2
NS = 16
CHUNK = 128


def _mlp_body(x_ref, w_ref, b_ref, o_ref):
    o_ref[...] = jnp.maximum(
        jnp.dot(x_ref[...], w_ref[...], preferred_element_type=jnp.float32)
        + b_ref[...],
        0.0,
    )


def _mlp(x2d, W, b):
    M, D = x2d.shape
    BM = 2000
    assert M % BM == 0
    return pl.pallas_call(
        _mlp_body,
        grid=(M // BM,),
        in_specs=[
            pl.BlockSpec((BM, D), lambda i: (i, 0)),
            pl.BlockSpec((D, D), lambda i: (0, 0)),
            pl.BlockSpec((1, D), lambda i: (0, 0)),
        ],
        out_specs=pl.BlockSpec((BM, D), lambda i: (i, 0)),
        out_shape=jax.ShapeDtypeStruct((M, D), jnp.float32),
    )(x2d, W, b.reshape(1, D))


@functools.partial(jax.jit, static_argnames=("B", "N", "D", "E_pad", "N_pad"))
def _sc_scatter(tx2d, src_g, dst_p, *, B, N, D, E_pad, N_pad):
    per_tile = E_pad // NS
    n_chunks = per_tile // CHUNK
    n_blocks = 4
    blk = n_chunks // n_blocks
    n_half = blk // 2
    rows_per_tile = N_pad // NS
    out_rows = (N // NS) // 8 * 8
    out_rem = N - out_rows * NS
    b_per_core = B // NC

    mesh = plsc.VectorSubcoreMesh(
        core_axis_name="c", subcore_axis_name="s",
        num_cores=NC, num_subcores=NS,
    )

    @functools.partial(
        pl.kernel,
        out_type=jax.ShapeDtypeStruct((B * N, D), jnp.float32),
        mesh=mesh,
        scratch_types=[
            pltpu.VMEM((blk, CHUNK), jnp.int32),
            pltpu.VMEM((blk, CHUNK), jnp.int32),
            pltpu.VMEM((CHUNK, D), jnp.float32),
            pltpu.VMEM((CHUNK, D), jnp.float32),
            pltpu.VMEM((16, D), jnp.float32),
            pltpu.VMEM_SHARED((N_pad, D), jnp.float32),
            pltpu.SemaphoreType.DMA,
            pltpu.SemaphoreType.DMA,
        ],
    )
    def sc_kernel(tx_hbm, srcg_hbm, dst_hbm, out_hbm,
                  src_blk, dst_blk, rows_a, rows_b, zeros_v, acc_sh,
                  gsem_a, gsem_b):
        c = lax.axis_index("c")
        s = lax.axis_index("s")

        zero16 = jnp.zeros((16,), jnp.float32)

        def zrow(i, carry):
            for j in range(D // 16):
                zeros_v[i, pl.ds(j * 16, 16)] = zero16
            return carry

        lax.fori_loop(0, 16, zrow, 0)

        for bi in range(b_per_core):
            b = c * b_per_core + bi

            def clr(j, carry):
                pltpu.sync_copy(
                    zeros_v,
                    acc_sh.at[pl.ds(s * rows_per_tile + j * 16, 16)])
                return carry

            lax.fori_loop(0, rows_per_tile // 16, clr, 0)
            plsc.subcore_barrier()

            def block_body(k, carry):
                pltpu.sync_copy(srcg_hbm.at[b * NS + s, k], src_blk)
                pltpu.sync_copy(dst_hbm.at[s, k], dst_blk)

                pltpu.async_copy(tx_hbm.at[src_blk.at[0]], rows_a, gsem_a)

                def chunk_pair(i, carry):
                    ga = 2 * i
                    gb = ga + 1
                    pltpu.async_copy(tx_hbm.at[src_blk.at[gb]], rows_b,
                                     gsem_b)
                    pltpu.make_async_copy(
                        tx_hbm.at[src_blk.at[ga]], rows_a, gsem_a).wait()
                    pltpu.sync_copy(rows_a, acc_sh.at[dst_blk.at[ga]],
                                    add=True)

                    @pl.when(i + 1 < n_half)
                    def _():
                        pltpu.async_copy(
                            tx_hbm.at[src_blk.at[ga + 2]], rows_a, gsem_a)

                    pltpu.make_async_copy(
                        tx_hbm.at[src_blk.at[gb]], rows_b, gsem_b).wait()
                    pltpu.sync_copy(rows_b, acc_sh.at[dst_blk.at[gb]],
                                    add=True)
                    return carry

                lax.fori_loop(0, n_half, chunk_pair, 0)
                return carry

            lax.fori_loop(0, n_blocks, block_body, 0)
            plsc.subcore_barrier()

            row_lo = s * out_rows
            pltpu.sync_copy(acc_sh.at[pl.ds(row_lo, out_rows)],
                            out_hbm.at[pl.ds(b * N + row_lo, out_rows)])
            if out_rem:
                @pl.when(s == NS - 1)
                def _():
                    lo = out_rows * NS
                    pltpu.sync_copy(
                        acc_sh.at[pl.ds(lo, out_rem)],
                        out_hbm.at[pl.ds(b * N + lo, out_rem)])
            plsc.subcore_barrier()

    return sc_kernel(tx2d, src_g, dst_p)


def kernel(X, edge_index, W, b):
    B, N, D = X.shape
    E = edge_index.shape[1]
    grain = NS * CHUNK * 8
    E_pad = ((E + grain - 1) // grain) * grain
    N_pad = ((N + 1 + NS * 64 - 1) // (NS * 64)) * (NS * 64)

    src = edge_index[0]
    dst = edge_index[1]
    pad = E_pad - E
    if pad:
        src = jnp.concatenate([src, jnp.zeros((pad,), jnp.int32)])
        dst = jnp.concatenate([dst, jnp.full((pad,), N, jnp.int32)])
    n_chunks = E_pad // NS // CHUNK
    src_g = (src[None, :]
             + (jnp.arange(B, dtype=jnp.int32) * N)[:, None])
    src_g = src_g.reshape(B * NS, 4, n_chunks // 4, CHUNK)
    dst3d = dst.reshape(NS, 4, n_chunks // 4, CHUNK)

    tx = _mlp(X.reshape(B * N, D), W, b)
    out2d = _sc_scatter(tx, src_g, dst3d, B=B, N=N, D=D,
                        E_pad=E_pad, N_pad=N_pad)
    return out2d.reshape(B, N, D)


# speedup vs baseline: 26.5444x; 1.0032x over previous
import functools

import jax
import jax.numpy as jnp
from jax import lax
from jax.experimental import pallas as pl
from jax.experimental.pallas import tpu as pltpu
from jax.experimental.pallas import tpu_sc as plsc

NC = 2
NS = 16
CHUNK = 128
SPLIT = 2


def _mlp_body(x_ref, w_ref, b_ref, o_ref):
    o_ref[...] = jnp.maximum(
        jnp.dot(x_ref[...], w_ref[...], preferred_element_type=jnp.float32)
        + b_ref[...],
        0.0,
    )


def _mlp(x2d, W, b):
    M, D = x2d.shape
    BM = 2000
    assert M % BM == 0
    return pl.pallas_call(
        _mlp_body,
        grid=(M // BM,),
        in_specs=[
            pl.BlockSpec((BM, D), lambda i: (i, 0)),
            pl.BlockSpec((D, D), lambda i: (0, 0)),
            pl.BlockSpec((1, D), lambda i: (0, 0)),
        ],
        out_specs=pl.BlockSpec((BM, D), lambda i: (i, 0)),
        out_shape=jax.ShapeDtypeStruct((M, D), jnp.float32),
    )(x2d, W, b.reshape(1, D))


@functools.partial(jax.jit, static_argnames=("B", "N", "D", "E_pad", "N_pad"))
def _sc_scatter(tx2d, src_g, dst_p, *, B, N, D, E_pad, N_pad):
    per_tile = E_pad // NS
    n_chunks = per_tile // CHUNK
    n_blocks = 4
    blk = n_chunks // n_blocks
    n_half = blk // 2
    rows_per_tile = N_pad // NS
    out_rows = (N // NS) // 8 * 8
    out_rem = N - out_rows * NS
    b_per_core = B // NC

    mesh = plsc.VectorSubcoreMesh(
        core_axis_name="c", subcore_axis_name="s",
        num_cores=NC, num_subcores=NS,
    )

    @functools.partial(
        pl.kernel,
        out_type=jax.ShapeDtypeStruct((B * N, D), jnp.float32),
        mesh=mesh,
        scratch_types=[
            pltpu.VMEM((blk, CHUNK), jnp.int32),
            pltpu.VMEM((blk, CHUNK), jnp.int32),
            pltpu.VMEM((CHUNK, D), jnp.float32),
            pltpu.VMEM((CHUNK, D), jnp.float32),
            pltpu.VMEM((16, D), jnp.float32),
            pltpu.VMEM_SHARED((N_pad, D), jnp.float32),
            pltpu.SemaphoreType.DMA,
            pltpu.SemaphoreType.DMA,
        ],
    )
    def sc_kernel(tx_hbm, srcg_hbm, dst_hbm, out_hbm,
                  src_blk, dst_blk, rows_a, rows_b, zeros_v, acc_sh,
                  gsem_a, gsem_b):
        c = lax.axis_index("c")
        s = lax.axis_index("s")

        zero16 = jnp.zeros((16,), jnp.float32)

        def zrow(i, carry):
            for j in range(D // 16):
                zeros_v[i, pl.ds(j * 16, 16)] = zero16
            return carry

        lax.fori_loop(0, 16, zrow, 0)

        for bi in range(b_per_core):
            b = c * b_per_core + bi

            def clr(j, carry):
                pltpu.sync_copy(
                    zeros_v,
                    acc_sh.at[pl.ds(s * rows_per_tile + j * 16, 16)])
                return carry

            lax.fori_loop(0, rows_per_tile // 16, clr, 0)
            plsc.subcore_barrier()

            def gather_chunk(g, rows, sem):
                for q in range(SPLIT):
                    lo = q * (CHUNK // SPLIT)
                    pltpu.async_copy(
                        tx_hbm.at[src_blk.at[g, pl.ds(lo, CHUNK // SPLIT)]],
                        rows.at[pl.ds(lo, CHUNK // SPLIT)], sem)

            def wait_chunk(g, rows, sem):
                for q in range(SPLIT):
                    lo = q * (CHUNK // SPLIT)
                    pltpu.make_async_copy(
                        tx_hbm.at[src_blk.at[g, pl.ds(lo, CHUNK // SPLIT)]],
                        rows.at[pl.ds(lo, CHUNK // SPLIT)], sem).wait()

            def block_body(k, carry):
                pltpu.sync_copy(srcg_hbm.at[b * NS + s, k], src_blk)
                pltpu.sync_copy(dst_hbm.at[s, k], dst_blk)

                gather_chunk(0, rows_a, gsem_a)

                def chunk_pair(i, carry):
                    ga = 2 * i
                    gb = ga + 1
                    gather_chunk(gb, rows_b, gsem_b)
                    wait_chunk(ga, rows_a, gsem_a)
                    pltpu.sync_copy(rows_a, acc_sh.at[dst_blk.at[ga]],
                                    add=True)

                    @pl.when(i + 1 < n_half)
                    def _():
                        gather_chunk(ga + 2, rows_a, gsem_a)

                    wait_chunk(gb, rows_b, gsem_b)
                    pltpu.sync_copy(rows_b, acc_sh.at[dst_blk.at[gb]],
                                    add=True)
                    return carry

                lax.fori_loop(0, n_half, chunk_pair, 0)
                return carry

            lax.fori_loop(0, n_blocks, block_body, 0)
            plsc.subcore_barrier()

            row_lo = s * out_rows
            pltpu.sync_copy(acc_sh.at[pl.ds(row_lo, out_rows)],
                            out_hbm.at[pl.ds(b * N + row_lo, out_rows)])
            if out_rem:
                @pl.when(s == NS - 1)
                def _():
                    lo = out_rows * NS
                    pltpu.sync_copy(
                        acc_sh.at[pl.ds(lo, out_rem)],
                        out_hbm.at[pl.ds(b * N + lo, out_rem)])
            plsc.subcore_barrier()

    return sc_kernel(tx2d, src_g, dst_p)


def kernel(X, edge_index, W, b):
    B, N, D = X.shape
    E = edge_index.shape[1]
    grain = NS * CHUNK * 8
    E_pad = ((E + grain - 1) // grain) * grain
    N_pad = ((N + 1 + NS * 64 - 1) // (NS * 64)) * (NS * 64)

    src = edge_index[0]
    dst = edge_index[1]
    pad = E_pad - E
    if pad:
        src = jnp.concatenate([src, jnp.zeros((pad,), jnp.int32)])
        dst = jnp.concatenate([dst, jnp.full((pad,), N, jnp.int32)])
    n_chunks = E_pad // NS // CHUNK
    src_g = (src[None, :]
             + (jnp.arange(B, dtype=jnp.int32) * N)[:, None])
    src_g = src_g.reshape(B * NS, 4, n_chunks // 4, CHUNK)
    dst3d = dst.reshape(NS, 4, n_chunks // 4, CHUNK)

    tx = _mlp(X.reshape(B * N, D), W, b)
    out2d = _sc_scatter(tx, src_g, dst3d, B=B, N=N, D=D,
                        E_pad=E_pad, N_pad=N_pad)
    return out2d.reshape(B, N, D)
